# baseline (speedup 1.0000x reference)
import jax
import jax.numpy as jnp
from jax.experimental import pallas as pl
from jax.experimental.pallas import tpu as pltpu

B = 32
C = 768
HWG = 24
L = HWG * HWG
TOPK = 288
ZDIM = 256
PATCH = 16
HWUP = HWG * PATCH

_F32 = jnp.float32


def _dot(a, b, dims, precision=jax.lax.Precision.HIGHEST):
    return jax.lax.dot_general(a, b, (dims, ((), ())),
                               precision=precision,
                               preferred_element_type=_F32)


def _main_body(feat_ref, W1_ref, b1_ref, w2_ref, b2_ref, Wp_ref, bp_ref,
               sample_ref, order_ref, sscore_ref, mask_ref, avg_ref, pred_ref):
    feat = feat_ref[0]
    W1 = W1_ref[...]
    b1 = b1_ref[...]
    w2 = w2_ref[...]
    b2 = b2_ref[0, 0]
    Wp = Wp_ref[...]
    bp = bp_ref[...]

    mu_row = jnp.mean(feat, axis=0, keepdims=True)
    avg_ref[0] = mu_row

    h = _dot(feat, W1, ((0,), (0,)), jax.lax.Precision.DEFAULT)
    h = jnp.maximum(h + b1, 0.0)
    logit_col = _dot(h, w2, ((1,), (0,)), jax.lax.Precision.DEFAULT)
    s_col = jax.nn.sigmoid(logit_col + b2)
    s_row = jnp.transpose(s_col)
    pred_ref[0] = s_row

    ii = jax.lax.broadcasted_iota(jnp.int32, (L, L), 0)
    jj = jax.lax.broadcasted_iota(jnp.int32, (L, L), 1)
    beats = (s_row > s_col) | ((s_row == s_col) & (jj < ii))
    rank_col = jnp.sum(beats.astype(_F32), axis=1, keepdims=True)

    r_row = jj.astype(_F32)
    Q = (rank_col == r_row).astype(_F32)

    idx_row = jax.lax.broadcasted_iota(jnp.int32, (1, L), 1).astype(_F32)
    order_row = _dot(idx_row, Q, ((1,), (0,)))
    order_ref[0] = order_row.astype(jnp.int32)
    sort_row = _dot(s_row, Q, ((1,), (0,)))
    sscore_ref[0] = sort_row
    mask_row = _dot(jnp.ones((1, TOPK), _F32), Q[:, :TOPK], ((1,), (1,)))
    mask_ref[0] = mask_row

    var_row = jnp.mean((feat - mu_row) ** 2, axis=0, keepdims=True)
    inv_row = jax.lax.rsqrt(var_row + 1e-5)
    zf = _dot(Wp, feat, ((0,), (0,)))
    wpsum_col = _dot(Wp, jnp.ones((1, C), _F32), ((0,), (1,)))
    znorm = (zf - wpsum_col * mu_row) * inv_row

    Qs = Q[:, :TOPK] * sort_row[:, :TOPK]
    sample_ref[0] = _dot(znorm, Qs, ((1,), (0,))) + bp


def _maps_body(pred_all_ref, pred_row_ref, mask_row_ref, bin_ref, smap_ref):
    pred_all = pred_all_ref[...]
    mn = jnp.min(pred_all)
    mx = jnp.max(pred_all)
    normed = (pred_row_ref[0] - mn) / jnp.maximum(mx - mn, 1e-5)
    mask_row = mask_row_ref[0]

    rg = jax.lax.broadcasted_iota(jnp.int32, (HWG, L), 0)
    ic = jax.lax.broadcasted_iota(jnp.int32, (HWG, L), 1)
    S = (ic // HWG == rg).astype(_F32)
    ir = jax.lax.broadcasted_iota(jnp.int32, (L, HWG), 0)
    cg = jax.lax.broadcasted_iota(jnp.int32, (L, HWG), 1)
    T = (ir % HWG == cg).astype(_F32)
    pp = jax.lax.broadcasted_iota(jnp.int32, (HWUP, HWG), 0)
    gg = jax.lax.broadcasted_iota(jnp.int32, (HWUP, HWG), 1)
    R = (pp // PATCH == gg).astype(_F32)

    def upsample(row):
        m2d = _dot(S * row, T, ((1,), (0,)))
        u1 = _dot(R, m2d, ((1,), (0,)))
        return _dot(u1, R, ((1,), (1,)))

    bin_ref[0, 0] = upsample(mask_row)
    smap_ref[0, 0] = upsample(normed)


def kernel(image_features, W1, b1, W2, b2, Wp, bp):
    feat = image_features.reshape(B, C, L)
    b1r = b1.reshape(1, C)
    w2r = W2
    b2r = b2.reshape(1, 1)
    bpc = bp.reshape(ZDIM, 1)

    const = lambda i: (0, 0)
    row = lambda i: (i, 0, 0)
    sample_h, order3, sscore3, mask3, avg3, pred3 = pl.pallas_call(
        _main_body,
        grid=(B,),
        in_specs=[
            pl.BlockSpec((1, C, L), row),
            pl.BlockSpec((C, C), const),
            pl.BlockSpec((1, C), const),
            pl.BlockSpec((C, 1), const),
            pl.BlockSpec((1, 1), const),
            pl.BlockSpec((C, ZDIM), const),
            pl.BlockSpec((ZDIM, 1), const),
        ],
        out_specs=[
            pl.BlockSpec((1, ZDIM, TOPK), row),
            pl.BlockSpec((1, 1, L), row),
            pl.BlockSpec((1, 1, L), row),
            pl.BlockSpec((1, 1, L), row),
            pl.BlockSpec((1, 1, L), row),
            pl.BlockSpec((1, 1, L), row),
        ],
        out_shape=[
            jax.ShapeDtypeStruct((B, ZDIM, TOPK), jnp.float32),
            jax.ShapeDtypeStruct((B, 1, L), jnp.int32),
            jax.ShapeDtypeStruct((B, 1, L), jnp.float32),
            jax.ShapeDtypeStruct((B, 1, L), jnp.float32),
            jax.ShapeDtypeStruct((B, 1, L), jnp.float32),
            jax.ShapeDtypeStruct((B, 1, L), jnp.float32),
        ],
        compiler_params=pltpu.CompilerParams(
            dimension_semantics=("arbitrary",)),
    )(feat, W1, b1r, w2r, b2r, Wp, bpc)

    binary_map, score_map = pl.pallas_call(
        _maps_body,
        grid=(B,),
        in_specs=[
            pl.BlockSpec((B, 1, L), lambda i: (0, 0, 0)),
            pl.BlockSpec((1, 1, L), row),
            pl.BlockSpec((1, 1, L), row),
        ],
        out_specs=[
            pl.BlockSpec((1, 1, HWUP, HWUP), lambda i: (i, 0, 0, 0)),
            pl.BlockSpec((1, 1, HWUP, HWUP), lambda i: (i, 0, 0, 0)),
        ],
        out_shape=[
            jax.ShapeDtypeStruct((B, 1, HWUP, HWUP), jnp.float32),
            jax.ShapeDtypeStruct((B, 1, HWUP, HWUP), jnp.float32),
        ],
        compiler_params=pltpu.CompilerParams(
            dimension_semantics=("arbitrary",)),
    )(pred3, pred3, mask3)

    order = order3.reshape(B, L)
    pred = pred3.reshape(B, L)
    return (sample_h, order[:, :TOPK], order[:, TOPK:], binary_map, score_map,
            mask3.reshape(B, L), sscore3.reshape(B, L)[:, :TOPK],
            avg3.reshape(B, HWG, HWG), pred)


# speedup vs baseline: 1.5724x; 1.5724x over previous
import jax
import jax.numpy as jnp
from jax.experimental import pallas as pl
from jax.experimental.pallas import tpu as pltpu

B = 32
C = 768
HWG = 24
L = HWG * HWG
TOPK = 288
ZDIM = 256
PATCH = 16
HWUP = HWG * PATCH

_F32 = jnp.float32


def _dot(a, b, dims, precision=jax.lax.Precision.HIGHEST):
    return jax.lax.dot_general(a, b, (dims, ((), ())),
                               precision=precision,
                               preferred_element_type=_F32)


def _main_body(feat_ref, W1_ref, b1_ref, w2_ref, b2_ref, Wp_ref, bp_ref,
               sample_ref, order_ref, sscore_ref, mask_ref, avg_ref, pred_ref):
    feat = feat_ref[0]
    W1 = W1_ref[...]
    b1 = b1_ref[...]
    w2 = w2_ref[...]
    b2 = b2_ref[0, 0]
    Wp = Wp_ref[...]
    bp = bp_ref[...]

    mu_row = jnp.mean(feat, axis=0, keepdims=True)
    avg_ref[0] = mu_row

    h = _dot(feat, W1, ((0,), (0,)), jax.lax.Precision.DEFAULT)
    h = jnp.maximum(h + b1, 0.0)
    logit_col = _dot(h, w2, ((1,), (0,)), jax.lax.Precision.DEFAULT)
    s_col = jax.nn.sigmoid(logit_col + b2)
    s_row = jnp.transpose(s_col)
    pred_ref[0] = s_row

    ii = jax.lax.broadcasted_iota(jnp.int32, (L, L), 0)
    jj = jax.lax.broadcasted_iota(jnp.int32, (L, L), 1)
    beats = (s_row > s_col) | ((s_row == s_col) & (jj < ii))
    rank_col = jnp.sum(beats.astype(_F32), axis=1, keepdims=True)

    r_row = jj.astype(_F32)
    Q = (rank_col == r_row).astype(_F32)

    idx_row = jax.lax.broadcasted_iota(jnp.int32, (1, L), 1).astype(_F32)
    stacked = jnp.concatenate([idx_row, s_row], axis=0)
    sorted2 = _dot(stacked, Q, ((1,), (0,)))
    order_ref[0] = sorted2[0:1].astype(jnp.int32)
    sort_row = sorted2[1:2]
    sscore_ref[0] = sort_row
    mask_row = jnp.transpose((rank_col < float(TOPK)).astype(_F32))
    mask_ref[0] = mask_row

    var_row = jnp.mean((feat - mu_row) ** 2, axis=0, keepdims=True)
    inv_row = jax.lax.rsqrt(var_row + 1e-5)
    zf = _dot(Wp, feat, ((0,), (0,)), jax.lax.Precision.DEFAULT)
    wpsum_col = _dot(Wp, jnp.ones((1, C), _F32), ((0,), (1,)))
    znorm = (zf - wpsum_col * mu_row) * inv_row

    znorm_q = znorm.astype(jnp.bfloat16).astype(_F32)
    gath = _dot(znorm_q, Q[:, :TOPK], ((1,), (0,)),
                jax.lax.Precision.DEFAULT)
    sample_ref[0] = gath * sort_row[:, :TOPK] + bp


def _maps_body(pred_all_ref, pred_row_ref, mask_row_ref, bin_ref, smap_ref):
    pred_all = pred_all_ref[...]
    mn = jnp.min(pred_all)
    mx = jnp.max(pred_all)
    normed = (pred_row_ref[0] - mn) / jnp.maximum(mx - mn, 1e-5)
    mask_row = mask_row_ref[0]

    rg = jax.lax.broadcasted_iota(jnp.int32, (HWG, L), 0)
    ic = jax.lax.broadcasted_iota(jnp.int32, (HWG, L), 1)
    S = (ic // HWG == rg).astype(_F32)
    ir = jax.lax.broadcasted_iota(jnp.int32, (L, HWG), 0)
    cg = jax.lax.broadcasted_iota(jnp.int32, (L, HWG), 1)
    T = (ir % HWG == cg).astype(_F32)
    pp = jax.lax.broadcasted_iota(jnp.int32, (HWUP, HWG), 0)
    gg = jax.lax.broadcasted_iota(jnp.int32, (HWUP, HWG), 1)
    R = (pp // PATCH == gg).astype(_F32)

    def upsample(row):
        d = jax.lax.Precision.DEFAULT
        m2d = _dot(S * row, T, ((1,), (0,)), d)
        u1 = _dot(R, m2d, ((1,), (0,)), d)
        return _dot(u1, R, ((1,), (1,)), d)

    bin_ref[0, 0] = upsample(mask_row)
    smap_ref[0, 0] = upsample(normed.astype(jnp.bfloat16).astype(_F32))


def kernel(image_features, W1, b1, W2, b2, Wp, bp):
    feat = image_features.reshape(B, C, L)
    b1r = b1.reshape(1, C)
    w2r = W2
    b2r = b2.reshape(1, 1)
    bpc = bp.reshape(ZDIM, 1)

    const = lambda i: (0, 0)
    row = lambda i: (i, 0, 0)
    sample_h, order3, sscore3, mask3, avg3, pred3 = pl.pallas_call(
        _main_body,
        grid=(B,),
        in_specs=[
            pl.BlockSpec((1, C, L), row),
            pl.BlockSpec((C, C), const),
            pl.BlockSpec((1, C), const),
            pl.BlockSpec((C, 1), const),
            pl.BlockSpec((1, 1), const),
            pl.BlockSpec((C, ZDIM), const),
            pl.BlockSpec((ZDIM, 1), const),
        ],
        out_specs=[
            pl.BlockSpec((1, ZDIM, TOPK), row),
            pl.BlockSpec((1, 1, L), row),
            pl.BlockSpec((1, 1, L), row),
            pl.BlockSpec((1, 1, L), row),
            pl.BlockSpec((1, 1, L), row),
            pl.BlockSpec((1, 1, L), row),
        ],
        out_shape=[
            jax.ShapeDtypeStruct((B, ZDIM, TOPK), jnp.float32),
            jax.ShapeDtypeStruct((B, 1, L), jnp.int32),
            jax.ShapeDtypeStruct((B, 1, L), jnp.float32),
            jax.ShapeDtypeStruct((B, 1, L), jnp.float32),
            jax.ShapeDtypeStruct((B, 1, L), jnp.float32),
            jax.ShapeDtypeStruct((B, 1, L), jnp.float32),
        ],
        compiler_params=pltpu.CompilerParams(
            dimension_semantics=("arbitrary",)),
    )(feat, W1, b1r, w2r, b2r, Wp, bpc)

    binary_map, score_map = pl.pallas_call(
        _maps_body,
        grid=(B,),
        in_specs=[
            pl.BlockSpec((B, 1, L), lambda i: (0, 0, 0)),
            pl.BlockSpec((1, 1, L), row),
            pl.BlockSpec((1, 1, L), row),
        ],
        out_specs=[
            pl.BlockSpec((1, 1, HWUP, HWUP), lambda i: (i, 0, 0, 0)),
            pl.BlockSpec((1, 1, HWUP, HWUP), lambda i: (i, 0, 0, 0)),
        ],
        out_shape=[
            jax.ShapeDtypeStruct((B, 1, HWUP, HWUP), jnp.float32),
            jax.ShapeDtypeStruct((B, 1, HWUP, HWUP), jnp.float32),
        ],
        compiler_params=pltpu.CompilerParams(
            dimension_semantics=("arbitrary",)),
    )(pred3, pred3, mask3)

    order = order3.reshape(B, L)
    pred = pred3.reshape(B, L)
    return (sample_h, order[:, :TOPK], order[:, TOPK:], binary_map, score_map,
            mask3.reshape(B, L), sscore3.reshape(B, L)[:, :TOPK],
            avg3.reshape(B, HWG, HWG), pred)


# speedup vs baseline: 1.6237x; 1.0327x over previous
import jax
import jax.numpy as jnp
from jax.experimental import pallas as pl
from jax.experimental.pallas import tpu as pltpu

B = 32
C = 768
HWG = 24
L = HWG * HWG
TOPK = 288
ZDIM = 256
PATCH = 16
HWUP = HWG * PATCH

_F32 = jnp.float32


def _dot(a, b, dims, precision=jax.lax.Precision.HIGHEST):
    return jax.lax.dot_general(a, b, (dims, ((), ())),
                               precision=precision,
                               preferred_element_type=_F32)


def _upsample_mats():
    rg = jax.lax.broadcasted_iota(jnp.int32, (HWG, L), 0)
    ic = jax.lax.broadcasted_iota(jnp.int32, (HWG, L), 1)
    S = (ic // HWG == rg).astype(_F32)
    ir = jax.lax.broadcasted_iota(jnp.int32, (L, HWG), 0)
    cg = jax.lax.broadcasted_iota(jnp.int32, (L, HWG), 1)
    T = (ir % HWG == cg).astype(_F32)
    pp = jax.lax.broadcasted_iota(jnp.int32, (HWUP, HWG), 0)
    gg = jax.lax.broadcasted_iota(jnp.int32, (HWUP, HWG), 1)
    R = (pp // PATCH == gg).astype(_F32)
    return S, T, R


def _upsample(S, T, R, row):
    d = jax.lax.Precision.DEFAULT
    m2d = _dot(S * row, T, ((1,), (0,)), d)
    u1 = _dot(R, m2d, ((1,), (0,)), d)
    return _dot(u1, R, ((1,), (1,)), d)


def _main_body(feat_ref, W1_ref, b1_ref, w2_ref, b2_ref, Wp_ref, bp_ref,
               sample_ref, order_ref, sscore_ref, mask_ref, avg_ref, pred_ref,
               bin_ref):
    feat = feat_ref[0]
    W1 = W1_ref[...]
    b1 = b1_ref[...]
    w2 = w2_ref[...]
    b2 = b2_ref[0, 0]
    Wp = Wp_ref[...]
    bp = bp_ref[...]

    mu_row = jnp.mean(feat, axis=0, keepdims=True)
    avg_ref[0] = mu_row

    h = _dot(feat, W1, ((0,), (0,)), jax.lax.Precision.DEFAULT)
    h = jnp.maximum(h + b1, 0.0)
    logit_col = _dot(h, w2, ((1,), (0,)), jax.lax.Precision.DEFAULT)
    s_col = jax.nn.sigmoid(logit_col + b2)
    s_row = jnp.transpose(s_col)
    pred_ref[0] = s_row

    ii = jax.lax.broadcasted_iota(jnp.int32, (L, L), 0)
    jj = jax.lax.broadcasted_iota(jnp.int32, (L, L), 1)
    beats = (s_row > s_col) | ((s_row == s_col) & (jj < ii))
    rank_col = _dot(beats.astype(_F32), jnp.ones((L, 1), _F32),
                    ((1,), (0,)), jax.lax.Precision.DEFAULT)

    r_row = jj.astype(_F32)
    Q = (rank_col == r_row).astype(_F32)

    idx_row = jax.lax.broadcasted_iota(jnp.int32, (1, L), 1)
    s_hi = s_row.astype(jnp.bfloat16).astype(_F32)
    s_lo = (s_row - s_hi).astype(jnp.bfloat16).astype(_F32)
    stacked = jnp.concatenate([
        (idx_row // 256).astype(_F32), (idx_row % 256).astype(_F32),
        s_hi, s_lo], axis=0)
    sorted4 = _dot(stacked, Q, ((1,), (0,)), jax.lax.Precision.DEFAULT)
    order_ref[0] = (sorted4[0:1] * 256.0 + sorted4[1:2]).astype(jnp.int32)
    sort_row = sorted4[2:3] + sorted4[3:4]
    sscore_ref[0] = sort_row
    mask_row = jnp.transpose((rank_col < float(TOPK)).astype(_F32))
    mask_ref[0] = mask_row
    S, T, R = _upsample_mats()
    bin_ref[0, 0] = _upsample(S, T, R, mask_row)

    var_row = jnp.mean((feat - mu_row) ** 2, axis=0, keepdims=True)
    inv_row = jax.lax.rsqrt(var_row + 1e-5)
    zf = _dot(Wp, feat, ((0,), (0,)), jax.lax.Precision.DEFAULT)
    wpsum_col = _dot(Wp, jnp.ones((1, C), _F32), ((0,), (1,)))
    znorm = (zf - wpsum_col * mu_row) * inv_row

    znorm_q = znorm.astype(jnp.bfloat16).astype(_F32)
    gath = _dot(znorm_q, Q[:, :TOPK], ((1,), (0,)),
                jax.lax.Precision.DEFAULT)
    sample_ref[0] = gath * sort_row[:, :TOPK] + bp


def _maps_body(pred_all_ref, pred_row_ref, smap_ref):
    pred_all = pred_all_ref[...]
    mn = jnp.min(pred_all)
    mx = jnp.max(pred_all)
    normed = (pred_row_ref[0] - mn) / jnp.maximum(mx - mn, 1e-5)
    S, T, R = _upsample_mats()
    smap_ref[0, 0] = _upsample(S, T, R,
                               normed.astype(jnp.bfloat16).astype(_F32))


def kernel(image_features, W1, b1, W2, b2, Wp, bp):
    feat = image_features.reshape(B, C, L)
    b1r = b1.reshape(1, C)
    w2r = W2
    b2r = b2.reshape(1, 1)
    bpc = bp.reshape(ZDIM, 1)

    const = lambda i: (0, 0)
    row = lambda i: (i, 0, 0)
    sample_h, order3, sscore3, mask3, avg3, pred3, binary_map = pl.pallas_call(
        _main_body,
        grid=(B,),
        in_specs=[
            pl.BlockSpec((1, C, L), row),
            pl.BlockSpec((C, C), const),
            pl.BlockSpec((1, C), const),
            pl.BlockSpec((C, 1), const),
            pl.BlockSpec((1, 1), const),
            pl.BlockSpec((C, ZDIM), const),
            pl.BlockSpec((ZDIM, 1), const),
        ],
        out_specs=[
            pl.BlockSpec((1, ZDIM, TOPK), row),
            pl.BlockSpec((1, 1, L), row),
            pl.BlockSpec((1, 1, L), row),
            pl.BlockSpec((1, 1, L), row),
            pl.BlockSpec((1, 1, L), row),
            pl.BlockSpec((1, 1, L), row),
            pl.BlockSpec((1, 1, HWUP, HWUP), lambda i: (i, 0, 0, 0)),
        ],
        out_shape=[
            jax.ShapeDtypeStruct((B, ZDIM, TOPK), jnp.float32),
            jax.ShapeDtypeStruct((B, 1, L), jnp.int32),
            jax.ShapeDtypeStruct((B, 1, L), jnp.float32),
            jax.ShapeDtypeStruct((B, 1, L), jnp.float32),
            jax.ShapeDtypeStruct((B, 1, L), jnp.float32),
            jax.ShapeDtypeStruct((B, 1, L), jnp.float32),
            jax.ShapeDtypeStruct((B, 1, HWUP, HWUP), jnp.float32),
        ],
        compiler_params=pltpu.CompilerParams(
            dimension_semantics=("parallel",)),
    )(feat, W1, b1r, w2r, b2r, Wp, bpc)

    score_map = pl.pallas_call(
        _maps_body,
        grid=(B,),
        in_specs=[
            pl.BlockSpec((B, 1, L), lambda i: (0, 0, 0)),
            pl.BlockSpec((1, 1, L), row),
        ],
        out_specs=pl.BlockSpec((1, 1, HWUP, HWUP), lambda i: (i, 0, 0, 0)),
        out_shape=jax.ShapeDtypeStruct((B, 1, HWUP, HWUP), jnp.float32),
        compiler_params=pltpu.CompilerParams(
            dimension_semantics=("parallel",)),
    )(pred3, pred3)

    order = order3.reshape(B, L)
    pred = pred3.reshape(B, L)
    return (sample_h, order[:, :TOPK], order[:, TOPK:], binary_map, score_map,
            mask3.reshape(B, L), sscore3.reshape(B, L)[:, :TOPK],
            avg3.reshape(B, HWG, HWG), pred)


# speedup vs baseline: 1.6246x; 1.0006x over previous
import jax
import jax.numpy as jnp
from jax.experimental import pallas as pl
from jax.experimental.pallas import tpu as pltpu

B = 32
C = 768
HWG = 24
L = HWG * HWG
TOPK = 288
ZDIM = 256
PATCH = 16
HWUP = HWG * PATCH

_F32 = jnp.float32


def _dot(a, b, dims, precision=jax.lax.Precision.HIGHEST):
    return jax.lax.dot_general(a, b, (dims, ((), ())),
                               precision=precision,
                               preferred_element_type=_F32)


def _upsample_mats():
    rg = jax.lax.broadcasted_iota(jnp.int32, (HWG, L), 0)
    ic = jax.lax.broadcasted_iota(jnp.int32, (HWG, L), 1)
    S = (ic // HWG == rg).astype(_F32)
    ir = jax.lax.broadcasted_iota(jnp.int32, (L, HWG), 0)
    cg = jax.lax.broadcasted_iota(jnp.int32, (L, HWG), 1)
    T = (ir % HWG == cg).astype(_F32)
    pp = jax.lax.broadcasted_iota(jnp.int32, (HWUP, HWG), 0)
    gg = jax.lax.broadcasted_iota(jnp.int32, (HWUP, HWG), 1)
    R = (pp // PATCH == gg).astype(_F32)
    return S, T, R


def _upsample(S, T, R, row):
    d = jax.lax.Precision.DEFAULT
    m2d = _dot(S * row, T, ((1,), (0,)), d)
    u1 = _dot(R, m2d, ((1,), (0,)), d)
    return _dot(u1, R, ((1,), (1,)), d)


def _main_body(feat_ref, W1_ref, b1_ref, w2_ref, b2_ref, Wp_ref, bp_ref,
               sample_ref, order_ref, sscore_ref, mask_ref, avg_ref, pred_ref,
               bin_ref):
    feat = feat_ref[0]
    W1 = W1_ref[...]
    b1 = b1_ref[...]
    w2 = w2_ref[...]
    b2 = b2_ref[0, 0]
    Wp = Wp_ref[...]
    bp = bp_ref[...]

    mu_row = jnp.mean(feat, axis=0, keepdims=True)
    avg_ref[0] = mu_row

    h = _dot(feat, W1, ((0,), (0,)), jax.lax.Precision.DEFAULT)
    h = jnp.maximum(h + b1, 0.0)
    logit_col = _dot(h, w2, ((1,), (0,)), jax.lax.Precision.DEFAULT)
    s_col = jax.nn.sigmoid(logit_col + b2)
    s_row = jnp.transpose(s_col)
    pred_ref[0] = s_row

    ii = jax.lax.broadcasted_iota(jnp.int32, (L, L), 0)
    jj = jax.lax.broadcasted_iota(jnp.int32, (L, L), 1)
    beats = ((s_row > s_col) | ((s_row == s_col) & (jj < ii))
             ).astype(jnp.bfloat16)
    rank_col = _dot(beats, jnp.ones((L, 1), jnp.bfloat16),
                    ((1,), (0,)), jax.lax.Precision.DEFAULT)

    r_row = jj.astype(_F32)
    Q = (rank_col == r_row).astype(jnp.bfloat16)

    idx_row = jax.lax.broadcasted_iota(jnp.int32, (1, L), 1)
    s_hi = s_row.astype(jnp.bfloat16)
    s_lo = (s_row - s_hi.astype(_F32)).astype(jnp.bfloat16)
    stacked = jnp.concatenate([
        (idx_row // 256).astype(jnp.bfloat16),
        (idx_row % 256).astype(jnp.bfloat16),
        s_hi, s_lo], axis=0)
    sorted4 = _dot(stacked, Q, ((1,), (0,)), jax.lax.Precision.DEFAULT)
    order_ref[0] = (sorted4[0:1] * 256.0 + sorted4[1:2]).astype(jnp.int32)
    sort_row = sorted4[2:3] + sorted4[3:4]
    sscore_ref[0] = sort_row
    mask_row = jnp.transpose((rank_col < float(TOPK)).astype(_F32))
    mask_ref[0] = mask_row
    S, T, R = _upsample_mats()
    bin_ref[0, 0] = _upsample(S, T, R, mask_row)

    var_row = jnp.mean((feat - mu_row) ** 2, axis=0, keepdims=True)
    inv_row = jax.lax.rsqrt(var_row + 1e-5)
    zf = _dot(Wp, feat, ((0,), (0,)), jax.lax.Precision.DEFAULT)
    wpsum_col = _dot(Wp, jnp.ones((1, C), _F32), ((0,), (1,)))
    znorm = (zf - wpsum_col * mu_row) * inv_row

    gath = _dot(znorm.astype(jnp.bfloat16), Q[:, :TOPK], ((1,), (0,)),
                jax.lax.Precision.DEFAULT)
    sample_ref[0] = gath * sort_row[:, :TOPK] + bp


def _maps_body(pred_all_ref, pred_row_ref, smap_ref):
    pred_all = pred_all_ref[...]
    mn = jnp.min(pred_all)
    mx = jnp.max(pred_all)
    normed = (pred_row_ref[0] - mn) / jnp.maximum(mx - mn, 1e-5)
    S, T, R = _upsample_mats()
    smap_ref[0, 0] = _upsample(S, T, R,
                               normed.astype(jnp.bfloat16).astype(_F32))


def kernel(image_features, W1, b1, W2, b2, Wp, bp):
    feat = image_features.reshape(B, C, L)
    b1r = b1.reshape(1, C)
    w2r = W2
    b2r = b2.reshape(1, 1)
    bpc = bp.reshape(ZDIM, 1)

    const = lambda i: (0, 0)
    row = lambda i: (i, 0, 0)
    sample_h, order3, sscore3, mask3, avg3, pred3, binary_map = pl.pallas_call(
        _main_body,
        grid=(B,),
        in_specs=[
            pl.BlockSpec((1, C, L), row),
            pl.BlockSpec((C, C), const),
            pl.BlockSpec((1, C), const),
            pl.BlockSpec((C, 1), const),
            pl.BlockSpec((1, 1), const),
            pl.BlockSpec((C, ZDIM), const),
            pl.BlockSpec((ZDIM, 1), const),
        ],
        out_specs=[
            pl.BlockSpec((1, ZDIM, TOPK), row),
            pl.BlockSpec((1, 1, L), row),
            pl.BlockSpec((1, 1, L), row),
            pl.BlockSpec((1, 1, L), row),
            pl.BlockSpec((1, 1, L), row),
            pl.BlockSpec((1, 1, L), row),
            pl.BlockSpec((1, 1, HWUP, HWUP), lambda i: (i, 0, 0, 0)),
        ],
        out_shape=[
            jax.ShapeDtypeStruct((B, ZDIM, TOPK), jnp.float32),
            jax.ShapeDtypeStruct((B, 1, L), jnp.int32),
            jax.ShapeDtypeStruct((B, 1, L), jnp.float32),
            jax.ShapeDtypeStruct((B, 1, L), jnp.float32),
            jax.ShapeDtypeStruct((B, 1, L), jnp.float32),
            jax.ShapeDtypeStruct((B, 1, L), jnp.float32),
            jax.ShapeDtypeStruct((B, 1, HWUP, HWUP), jnp.float32),
        ],
        compiler_params=pltpu.CompilerParams(
            dimension_semantics=("parallel",)),
    )(feat, W1, b1r, w2r, b2r, Wp, bpc)

    score_map = pl.pallas_call(
        _maps_body,
        grid=(B,),
        in_specs=[
            pl.BlockSpec((B, 1, L), lambda i: (0, 0, 0)),
            pl.BlockSpec((1, 1, L), row),
        ],
        out_specs=pl.BlockSpec((1, 1, HWUP, HWUP), lambda i: (i, 0, 0, 0)),
        out_shape=jax.ShapeDtypeStruct((B, 1, HWUP, HWUP), jnp.float32),
        compiler_params=pltpu.CompilerParams(
            dimension_semantics=("parallel",)),
    )(pred3, pred3)

    order = order3.reshape(B, L)
    pred = pred3.reshape(B, L)
    return (sample_h, order[:, :TOPK], order[:, TOPK:], binary_map, score_map,
            mask3.reshape(B, L), sscore3.reshape(B, L)[:, :TOPK],
            avg3.reshape(B, HWG, HWG), pred)


# speedup vs baseline: 1.6481x; 1.0144x over previous
import jax
import jax.numpy as jnp
from jax.experimental import pallas as pl
from jax.experimental.pallas import tpu as pltpu

B = 32
C = 768
HWG = 24
L = HWG * HWG
TOPK = 288
ZDIM = 256
PATCH = 16
HWUP = HWG * PATCH

_F32 = jnp.float32


def _dot(a, b, dims, precision=jax.lax.Precision.HIGHEST):
    return jax.lax.dot_general(a, b, (dims, ((), ())),
                               precision=precision,
                               preferred_element_type=_F32)


def _upsample_mats():
    rg = jax.lax.broadcasted_iota(jnp.int32, (HWG, L), 0)
    ic = jax.lax.broadcasted_iota(jnp.int32, (HWG, L), 1)
    S = (ic // HWG == rg).astype(_F32)
    ir = jax.lax.broadcasted_iota(jnp.int32, (L, HWG), 0)
    cg = jax.lax.broadcasted_iota(jnp.int32, (L, HWG), 1)
    T = (ir % HWG == cg).astype(_F32)
    pp = jax.lax.broadcasted_iota(jnp.int32, (HWUP, HWG), 0)
    gg = jax.lax.broadcasted_iota(jnp.int32, (HWUP, HWG), 1)
    R = (pp // PATCH == gg).astype(_F32)
    return S, T, R


def _upsample(S, T, R, row):
    d = jax.lax.Precision.DEFAULT
    m2d = _dot(S * row, T, ((1,), (0,)), d)
    u1 = _dot(R, m2d, ((1,), (0,)), d)
    return _dot(u1, R, ((1,), (1,)), d)


BB = 2


def _main_body(feat_ref, W1_ref, b1_ref, w2_ref, b2_ref, Wp_ref, bp_ref,
               sample_ref, order_ref, sscore_ref, mask_ref, avg_ref, pred_ref,
               bin_ref):
    for bb in range(BB):
        _one_batch(bb, feat_ref, W1_ref, b1_ref, w2_ref, b2_ref, Wp_ref,
                   bp_ref, sample_ref, order_ref, sscore_ref, mask_ref,
                   avg_ref, pred_ref, bin_ref)


def _one_batch(bb, feat_ref, W1_ref, b1_ref, w2_ref, b2_ref, Wp_ref, bp_ref,
               sample_ref, order_ref, sscore_ref, mask_ref, avg_ref, pred_ref,
               bin_ref):
    feat = feat_ref[bb]
    W1 = W1_ref[...]
    b1 = b1_ref[...]
    w2 = w2_ref[...]
    b2 = b2_ref[0, 0]
    Wp = Wp_ref[...]
    bp = bp_ref[...]

    mu_row = jnp.mean(feat, axis=0, keepdims=True)
    avg_ref[bb] = mu_row

    h = _dot(feat, W1, ((0,), (0,)), jax.lax.Precision.DEFAULT)
    h = jnp.maximum(h + b1, 0.0)
    logit_col = _dot(h, w2, ((1,), (0,)), jax.lax.Precision.DEFAULT)
    s_col = jax.nn.sigmoid(logit_col + b2)
    s_row = jnp.transpose(s_col)
    pred_ref[bb] = s_row

    ii = jax.lax.broadcasted_iota(jnp.int32, (L, L), 0)
    jj = jax.lax.broadcasted_iota(jnp.int32, (L, L), 1)
    beats = ((s_row > s_col) | ((s_row == s_col) & (jj < ii))
             ).astype(jnp.bfloat16)
    rank_col = _dot(beats, jnp.ones((L, 1), jnp.bfloat16),
                    ((1,), (0,)), jax.lax.Precision.DEFAULT)

    r_row = jj.astype(_F32)
    Q = (rank_col == r_row).astype(jnp.bfloat16)

    idx_row = jax.lax.broadcasted_iota(jnp.int32, (1, L), 1)
    s_hi = s_row.astype(jnp.bfloat16)
    s_lo = (s_row - s_hi.astype(_F32)).astype(jnp.bfloat16)
    stacked = jnp.concatenate([
        (idx_row // 256).astype(jnp.bfloat16),
        (idx_row % 256).astype(jnp.bfloat16),
        s_hi, s_lo], axis=0)
    sorted4 = _dot(stacked, Q, ((1,), (0,)), jax.lax.Precision.DEFAULT)
    order_ref[bb] = (sorted4[0:1] * 256.0 + sorted4[1:2]).astype(jnp.int32)
    sort_row = sorted4[2:3] + sorted4[3:4]
    sscore_ref[bb] = sort_row
    mask_row = jnp.transpose((rank_col < float(TOPK)).astype(_F32))
    mask_ref[bb] = mask_row
    S, T, R = _upsample_mats()
    bin_ref[bb, 0] = _upsample(S, T, R, mask_row)

    var_row = jnp.mean((feat - mu_row) ** 2, axis=0, keepdims=True)
    inv_row = jax.lax.rsqrt(var_row + 1e-5)
    zf = _dot(Wp, feat, ((0,), (0,)), jax.lax.Precision.DEFAULT)
    wpsum_col = _dot(Wp, jnp.ones((1, C), _F32), ((0,), (1,)))
    znorm = (zf - wpsum_col * mu_row) * inv_row

    gath = _dot(znorm.astype(jnp.bfloat16), Q[:, :TOPK], ((1,), (0,)),
                jax.lax.Precision.DEFAULT)
    sample_ref[bb] = gath * sort_row[:, :TOPK] + bp


def _maps_body(pred_all_ref, pred_row_ref, smap_ref):
    pred_all = pred_all_ref[...]
    mn = jnp.min(pred_all)
    mx = jnp.max(pred_all)
    normed = (pred_row_ref[0] - mn) / jnp.maximum(mx - mn, 1e-5)
    S, T, R = _upsample_mats()
    smap_ref[0, 0] = _upsample(S, T, R,
                               normed.astype(jnp.bfloat16).astype(_F32))


def kernel(image_features, W1, b1, W2, b2, Wp, bp):
    feat = image_features.reshape(B, C, L)
    b1r = b1.reshape(1, C)
    w2r = W2
    b2r = b2.reshape(1, 1)
    bpc = bp.reshape(ZDIM, 1)

    const = lambda i: (0, 0)
    row = lambda i: (i, 0, 0)
    sample_h, order3, sscore3, mask3, avg3, pred3, binary_map = pl.pallas_call(
        _main_body,
        grid=(B // BB,),
        in_specs=[
            pl.BlockSpec((BB, C, L), row),
            pl.BlockSpec((C, C), const),
            pl.BlockSpec((1, C), const),
            pl.BlockSpec((C, 1), const),
            pl.BlockSpec((1, 1), const),
            pl.BlockSpec((C, ZDIM), const),
            pl.BlockSpec((ZDIM, 1), const),
        ],
        out_specs=[
            pl.BlockSpec((BB, ZDIM, TOPK), row),
            pl.BlockSpec((BB, 1, L), row),
            pl.BlockSpec((BB, 1, L), row),
            pl.BlockSpec((BB, 1, L), row),
            pl.BlockSpec((BB, 1, L), row),
            pl.BlockSpec((BB, 1, L), row),
            pl.BlockSpec((BB, 1, HWUP, HWUP), lambda i: (i, 0, 0, 0)),
        ],
        out_shape=[
            jax.ShapeDtypeStruct((B, ZDIM, TOPK), jnp.float32),
            jax.ShapeDtypeStruct((B, 1, L), jnp.int32),
            jax.ShapeDtypeStruct((B, 1, L), jnp.float32),
            jax.ShapeDtypeStruct((B, 1, L), jnp.float32),
            jax.ShapeDtypeStruct((B, 1, L), jnp.float32),
            jax.ShapeDtypeStruct((B, 1, L), jnp.float32),
            jax.ShapeDtypeStruct((B, 1, HWUP, HWUP), jnp.float32),
        ],
        compiler_params=pltpu.CompilerParams(
            dimension_semantics=("parallel",)),
    )(feat, W1, b1r, w2r, b2r, Wp, bpc)

    score_map = pl.pallas_call(
        _maps_body,
        grid=(B,),
        in_specs=[
            pl.BlockSpec((B, 1, L), lambda i: (0, 0, 0)),
            pl.BlockSpec((1, 1, L), row),
        ],
        out_specs=pl.BlockSpec((1, 1, HWUP, HWUP), lambda i: (i, 0, 0, 0)),
        out_shape=jax.ShapeDtypeStruct((B, 1, HWUP, HWUP), jnp.float32),
        compiler_params=pltpu.CompilerParams(
            dimension_semantics=("parallel",)),
    )(pred3, pred3)

    order = order3.reshape(B, L)
    pred = pred3.reshape(B, L)
    return (sample_h, order[:, :TOPK], order[:, TOPK:], binary_map, score_map,
            mask3.reshape(B, L), sscore3.reshape(B, L)[:, :TOPK],
            avg3.reshape(B, HWG, HWG), pred)


# speedup vs baseline: 1.6505x; 1.0015x over previous
import jax
import jax.numpy as jnp
from jax.experimental import pallas as pl
from jax.experimental.pallas import tpu as pltpu

B = 32
C = 768
HWG = 24
L = HWG * HWG
TOPK = 288
ZDIM = 256
PATCH = 16
HWUP = HWG * PATCH

_F32 = jnp.float32


def _dot(a, b, dims, precision=jax.lax.Precision.HIGHEST):
    return jax.lax.dot_general(a, b, (dims, ((), ())),
                               precision=precision,
                               preferred_element_type=_F32)


def _upsample_mats():
    rg = jax.lax.broadcasted_iota(jnp.int32, (HWG, L), 0)
    ic = jax.lax.broadcasted_iota(jnp.int32, (HWG, L), 1)
    S = (ic // HWG == rg).astype(_F32)
    ir = jax.lax.broadcasted_iota(jnp.int32, (L, HWG), 0)
    cg = jax.lax.broadcasted_iota(jnp.int32, (L, HWG), 1)
    T = (ir % HWG == cg).astype(_F32)
    pp = jax.lax.broadcasted_iota(jnp.int32, (HWUP, HWG), 0)
    gg = jax.lax.broadcasted_iota(jnp.int32, (HWUP, HWG), 1)
    R = (pp // PATCH == gg).astype(_F32)
    return S, T, R


def _upsample(S, T, R, row):
    d = jax.lax.Precision.DEFAULT
    m2d = _dot(S * row, T, ((1,), (0,)), d)
    u1 = _dot(R, m2d, ((1,), (0,)), d)
    return _dot(u1, R, ((1,), (1,)), d)


BB = 4


def _main_body(feat_ref, W1_ref, b1_ref, w2_ref, b2_ref, Wp_ref, bp_ref,
               sample_ref, order_ref, sscore_ref, mask_ref, avg_ref, pred_ref,
               bin_ref):
    for bb in range(BB):
        _one_batch(bb, feat_ref, W1_ref, b1_ref, w2_ref, b2_ref, Wp_ref,
                   bp_ref, sample_ref, order_ref, sscore_ref, mask_ref,
                   avg_ref, pred_ref, bin_ref)


def _one_batch(bb, feat_ref, W1_ref, b1_ref, w2_ref, b2_ref, Wp_ref, bp_ref,
               sample_ref, order_ref, sscore_ref, mask_ref, avg_ref, pred_ref,
               bin_ref):
    feat = feat_ref[bb]
    W1 = W1_ref[...]
    b1 = b1_ref[...]
    w2 = w2_ref[...]
    b2 = b2_ref[0, 0]
    Wp = Wp_ref[...]
    bp = bp_ref[...]

    mu_row = jnp.mean(feat, axis=0, keepdims=True)
    avg_ref[bb] = mu_row

    h = _dot(feat, W1, ((0,), (0,)), jax.lax.Precision.DEFAULT)
    h = jnp.maximum(h + b1, 0.0)
    logit_col = _dot(h, w2, ((1,), (0,)), jax.lax.Precision.DEFAULT)
    s_col = jax.nn.sigmoid(logit_col + b2)
    s_row = jnp.transpose(s_col)
    pred_ref[bb] = s_row

    ii = jax.lax.broadcasted_iota(jnp.int32, (L, L), 0)
    jj = jax.lax.broadcasted_iota(jnp.int32, (L, L), 1)
    beats = ((s_row > s_col) | ((s_row == s_col) & (jj < ii))
             ).astype(jnp.bfloat16)
    rank_col = _dot(beats, jnp.ones((L, 1), jnp.bfloat16),
                    ((1,), (0,)), jax.lax.Precision.DEFAULT)

    r_row = jj.astype(_F32)
    Q = (rank_col == r_row).astype(jnp.bfloat16)

    idx_row = jax.lax.broadcasted_iota(jnp.int32, (1, L), 1)
    s_hi = s_row.astype(jnp.bfloat16)
    s_lo = (s_row - s_hi.astype(_F32)).astype(jnp.bfloat16)
    stacked = jnp.concatenate([
        (idx_row // 256).astype(jnp.bfloat16),
        (idx_row % 256).astype(jnp.bfloat16),
        s_hi, s_lo], axis=0)
    sorted4 = _dot(stacked, Q, ((1,), (0,)), jax.lax.Precision.DEFAULT)
    order_ref[bb] = (sorted4[0:1] * 256.0 + sorted4[1:2]).astype(jnp.int32)
    sort_row = sorted4[2:3] + sorted4[3:4]
    sscore_ref[bb] = sort_row
    mask_row = jnp.transpose((rank_col < float(TOPK)).astype(_F32))
    mask_ref[bb] = mask_row
    S, T, R = _upsample_mats()
    bin_ref[bb, 0] = _upsample(S, T, R, mask_row)

    var_row = jnp.mean(feat * feat, axis=0, keepdims=True) - mu_row * mu_row
    inv_row = jax.lax.rsqrt(var_row + 1e-5)
    zf = _dot(Wp, feat, ((0,), (0,)), jax.lax.Precision.DEFAULT)
    wpsum_col = _dot(Wp, jnp.ones((1, C), _F32), ((0,), (1,)))
    znorm = (zf - wpsum_col * mu_row) * inv_row

    gath = _dot(znorm.astype(jnp.bfloat16), Q[:, :TOPK], ((1,), (0,)),
                jax.lax.Precision.DEFAULT)
    sample_ref[bb] = gath * sort_row[:, :TOPK] + bp


def _maps_body(pred_all_ref, pred_row_ref, smap_ref):
    pred_all = pred_all_ref[...]
    mn = jnp.min(pred_all)
    mx = jnp.max(pred_all)
    normed = (pred_row_ref[0] - mn) / jnp.maximum(mx - mn, 1e-5)
    S, T, R = _upsample_mats()
    smap_ref[0, 0] = _upsample(S, T, R,
                               normed.astype(jnp.bfloat16).astype(_F32))


def kernel(image_features, W1, b1, W2, b2, Wp, bp):
    feat = image_features.reshape(B, C, L)
    b1r = b1.reshape(1, C)
    w2r = W2
    b2r = b2.reshape(1, 1)
    bpc = bp.reshape(ZDIM, 1)

    const = lambda i: (0, 0)
    row = lambda i: (i, 0, 0)
    sample_h, order3, sscore3, mask3, avg3, pred3, binary_map = pl.pallas_call(
        _main_body,
        grid=(B // BB,),
        in_specs=[
            pl.BlockSpec((BB, C, L), row),
            pl.BlockSpec((C, C), const),
            pl.BlockSpec((1, C), const),
            pl.BlockSpec((C, 1), const),
            pl.BlockSpec((1, 1), const),
            pl.BlockSpec((C, ZDIM), const),
            pl.BlockSpec((ZDIM, 1), const),
        ],
        out_specs=[
            pl.BlockSpec((BB, ZDIM, TOPK), row),
            pl.BlockSpec((BB, 1, L), row),
            pl.BlockSpec((BB, 1, L), row),
            pl.BlockSpec((BB, 1, L), row),
            pl.BlockSpec((BB, 1, L), row),
            pl.BlockSpec((BB, 1, L), row),
            pl.BlockSpec((BB, 1, HWUP, HWUP), lambda i: (i, 0, 0, 0)),
        ],
        out_shape=[
            jax.ShapeDtypeStruct((B, ZDIM, TOPK), jnp.float32),
            jax.ShapeDtypeStruct((B, 1, L), jnp.int32),
            jax.ShapeDtypeStruct((B, 1, L), jnp.float32),
            jax.ShapeDtypeStruct((B, 1, L), jnp.float32),
            jax.ShapeDtypeStruct((B, 1, L), jnp.float32),
            jax.ShapeDtypeStruct((B, 1, L), jnp.float32),
            jax.ShapeDtypeStruct((B, 1, HWUP, HWUP), jnp.float32),
        ],
        compiler_params=pltpu.CompilerParams(
            dimension_semantics=("parallel",)),
    )(feat, W1, b1r, w2r, b2r, Wp, bpc)

    score_map = pl.pallas_call(
        _maps_body,
        grid=(B,),
        in_specs=[
            pl.BlockSpec((B, 1, L), lambda i: (0, 0, 0)),
            pl.BlockSpec((1, 1, L), row),
        ],
        out_specs=pl.BlockSpec((1, 1, HWUP, HWUP), lambda i: (i, 0, 0, 0)),
        out_shape=jax.ShapeDtypeStruct((B, 1, HWUP, HWUP), jnp.float32),
        compiler_params=pltpu.CompilerParams(
            dimension_semantics=("parallel",)),
    )(pred3, pred3)

    order = order3.reshape(B, L)
    pred = pred3.reshape(B, L)
    return (sample_h, order[:, :TOPK], order[:, TOPK:], binary_map, score_map,
            mask3.reshape(B, L), sscore3.reshape(B, L)[:, :TOPK],
            avg3.reshape(B, HWG, HWG), pred)
